# baseline (speedup 1.0000x reference)
import jax
import jax.numpy as jnp
from jax.experimental import pallas as pl

N = 10000
B = 16
HIDDEN = 32
NUM_LAYERS = 2
K = 2


def _spmm(row, col, val, x):
    return jax.ops.segment_sum(val[:, None] * x[col], row, num_segments=N)


def _gconv(x_cat, supports, W, b):
    input_size = x_cat.shape[2]
    x0 = jnp.transpose(x_cat, (1, 2, 0)).reshape(N, input_size * B)
    xs = [x0]
    for (row, col, val) in supports:
        x1 = _spmm(row, col, val, x0)
        xs.append(x1)
        for _k in range(2, K + 1):
            x2 = 2.0 * _spmm(row, col, val, x1) - x0
            xs.append(x2)
            x1, x0 = x2, x1
    num_matrices = len(xs)
    x = jnp.stack(xs, axis=0).reshape(num_matrices, N, input_size, B)
    x = jnp.transpose(x, (3, 1, 2, 0)).reshape(B * N, input_size * num_matrices)
    x = x @ W + b
    return x.reshape(B, N * W.shape[1])


def _dcgru_cell(x, hx, supports, Wg, bg, Wc, bc):
    inp = x.reshape(B, N, -1)
    h = hx.reshape(B, N, HIDDEN)
    value = jax.nn.sigmoid(_gconv(jnp.concatenate([inp, h], axis=2), supports, Wg, bg))
    value = value.reshape(B, N, 2 * HIDDEN)
    r = value[:, :, :HIDDEN].reshape(B, N * HIDDEN)
    u = value[:, :, HIDDEN:].reshape(B, N * HIDDEN)
    rh = (r * hx).reshape(B, N, HIDDEN)
    c = jnp.tanh(_gconv(jnp.concatenate([inp, rh], axis=2), supports, Wc, bc))
    return u * hx + (1.0 - u) * c


def _copy_kernel(x_ref, o_ref):
    o_ref[...] = x_ref[...]


def kernel(inputs, hidden_state, edge_src, edge_dst, edge_w,
           Wg0, bg0, Wc0, bc0, Wg1, bg1, Wc1, bc1):
    deg_out = jax.ops.segment_sum(edge_w, edge_src, num_segments=N)
    inv_out = jnp.where(deg_out > 0, 1.0 / deg_out, 0.0)
    s1 = (edge_dst, edge_src, edge_w * inv_out[edge_src])
    deg_in = jax.ops.segment_sum(edge_w, edge_dst, num_segments=N)
    inv_in = jnp.where(deg_in > 0, 1.0 / deg_in, 0.0)
    s2 = (edge_src, edge_dst, edge_w * inv_in[edge_dst])
    supports = [s1, s2]
    params = [(Wg0, bg0, Wc0, bc0), (Wg1, bg1, Wc1, bc1)]
    output = inputs
    hidden_states = []
    for l in range(NUM_LAYERS):
        Wg, bg, Wc, bc = params[l]
        nh = _dcgru_cell(output, hidden_state[l], supports, Wg, bg, Wc, bc)
        hidden_states.append(nh)
        output = nh
    out = pl.pallas_call(
        _copy_kernel,
        out_shape=jax.ShapeDtypeStruct(output.shape, output.dtype),
    )(output)
    return (out, jnp.stack(hidden_states))


# speedup vs baseline: 2.2736x; 2.2736x over previous
import functools

import jax
import jax.numpy as jnp
from jax import lax
from jax.experimental import pallas as pl
from jax.experimental.pallas import tpu as pltpu
from jax.experimental.pallas import tpu_sc as plsc

N = 10000
E = 160000
B = 16
INPUT_DIM = 2
HIDDEN = 32
NUM_LAYERS = 2
K = 2
M = 2 * K + 1

NUM_TILES = 16
NUM_CORES = 2
KB = 128
NB = -(-(E // NUM_TILES) // KB)
EP_TILE = NB * KB
E_PAD = EP_TILE * NUM_TILES

N_PAD = 10240
STRIPE = N_PAD // NUM_TILES


def _splat(i):
    return jnp.full((16,), i, jnp.int32)

FC = 128

R = N_PAD * B
BR = 2048


def _chain_body(S, x_hbm, srci_hbm, dsti_hbm, v1_hbm, v2_hbm, zeros_hbm,
                t1_hbm, t2_hbm, t3_hbm, t4_hbm,
                src_v, dst_v, valb_v, buf_v, acc_sh, sem):
    c = lax.axis_index("c")
    s = lax.axis_index("s")
    spc = S // NUM_CORES
    pltpu.sync_copy(srci_hbm.at[s], src_v)
    pltpu.sync_copy(dsti_hbm.at[s], dst_v)
    v1_t = v1_hbm.at[s]
    v2_t = v2_hbm.at[s]
    stages = [
        (x_hbm, src_v, dst_v, v1_t, False, t1_hbm),
        (t1_hbm, src_v, dst_v, v1_t, True, t2_hbm),
        (t1_hbm, dst_v, src_v, v2_t, False, t3_hbm),
        (t3_hbm, dst_v, src_v, v2_t, True, t4_hbm),
    ]
    for p in range(spc):
        sl = c * spc + p
        for x_in, gat_v, scat_v, v_t, dbl, out_hbm in stages:
            pltpu.sync_copy(zeros_hbm.at[pl.ds(s * STRIPE, STRIPE)],
                            acc_sh.at[pl.ds(s * STRIPE, STRIPE)])
            plsc.subcore_barrier()
            x_sl = x_in.at[sl]

            def batch_body(b, carry):
                pltpu.async_copy(x_sl.at[gat_v.at[b]], buf_v, sem).wait()
                pltpu.sync_copy(v_t.at[pl.ds(b * KB, KB)], valb_v)

                def group_body(g, carry2):
                    vk = valb_v[pl.ds(g * 16, 16)]
                    if dbl:
                        vk = vk + vk
                    for rr in range(16):
                        v = vk.at[_splat(rr)].get(mode="promise_in_bounds")
                        r = g * 16 + rr
                        for j in range(FC // 16):
                            sl16 = pl.ds(j * 16, 16)
                            buf_v[r, sl16] = buf_v[r, sl16] * v
                    return carry2

                lax.fori_loop(0, KB // 16, group_body, 0)
                pltpu.sync_copy(buf_v, acc_sh.at[scat_v.at[b]], add=True)
                return carry

            lax.fori_loop(0, NB, batch_body, 0)
            plsc.subcore_barrier()
            pltpu.sync_copy(acc_sh.at[pl.ds(s * STRIPE, STRIPE)],
                            out_hbm.at[sl, pl.ds(s * STRIPE, STRIPE)])
            plsc.subcore_barrier()


@functools.cache
def _make_chain(S):
    body = functools.partial(_chain_body, S)
    term = jax.ShapeDtypeStruct((S, N_PAD, FC), jnp.float32)
    return pl.kernel(
        body,
        out_type=(term, term, term, term),
        mesh=plsc.VectorSubcoreMesh(core_axis_name="c", subcore_axis_name="s",
                                    num_cores=NUM_CORES,
                                    num_subcores=NUM_TILES),
        scratch_types=[
            pltpu.VMEM((NB, KB), jnp.int32),
            pltpu.VMEM((NB, KB), jnp.int32),
            pltpu.VMEM((KB,), jnp.float32),
            pltpu.VMEM((KB, FC), jnp.float32),
            pltpu.VMEM_SHARED((N_PAD, FC), jnp.float32),
            pltpu.SemaphoreType.DMA,
        ],
    )


def _gate_body(n_inp, *refs):
    ti = refs[0:n_inp]
    th = refs[n_inp:n_inp + M]
    wi, wh, bg, h = refs[n_inp + M:n_inp + M + 4]
    rh_o, u_o = refs[n_inp + M + 4:]
    acc = jnp.zeros((BR, 2 * HIDDEN), jnp.float32) + bg[...]
    for m in range(n_inp):
        acc += jnp.dot(ti[m][...], wi[m], preferred_element_type=jnp.float32)
    for m in range(M):
        acc += jnp.dot(th[m][...], wh[m], preferred_element_type=jnp.float32)
    value = jax.nn.sigmoid(acc)
    r = value[:, :HIDDEN]
    u = value[:, HIDDEN:]
    rh_o[...] = r * h[...]
    u_o[...] = u


def _cand_body(n_inp, *refs):
    ti = refs[0:n_inp]
    trh = refs[n_inp:n_inp + M]
    wi, wh, bc, u, hx = refs[n_inp + M:n_inp + M + 5]
    hn_o = refs[n_inp + M + 5]
    acc = jnp.zeros((BR, HIDDEN), jnp.float32) + bc[...]
    for m in range(n_inp):
        acc += jnp.dot(ti[m][...], wi[m], preferred_element_type=jnp.float32)
    for m in range(M):
        acc += jnp.dot(trh[m][...], wh[m], preferred_element_type=jnp.float32)
    c = jnp.tanh(acc)
    uu = u[...]
    hn_o[...] = uu * hx[...] + (1.0 - uu) * c


def _row_spec(cols):
    return pl.BlockSpec((BR, cols), lambda i: (i, 0))


def _full_spec(shape):
    nd = len(shape)
    return pl.BlockSpec(shape, lambda i: (0,) * nd)


@functools.cache
def _make_gate(n_inp, ci):
    in_specs = (
        [_row_spec(ci)] * n_inp
        + [_row_spec(HIDDEN)] * M
        + [_full_spec((n_inp, ci, 2 * HIDDEN)),
           _full_spec((M, HIDDEN, 2 * HIDDEN)),
           _full_spec((1, 2 * HIDDEN)),
           _row_spec(HIDDEN)]
    )
    return pl.pallas_call(
        functools.partial(_gate_body, n_inp),
        grid=(R // BR,),
        in_specs=in_specs,
        out_specs=[_row_spec(HIDDEN)] * 2,
        out_shape=[jax.ShapeDtypeStruct((R, HIDDEN), jnp.float32)] * 2,
    )


@functools.cache
def _make_cand(n_inp, ci):
    in_specs = (
        [_row_spec(ci)] * n_inp
        + [_row_spec(HIDDEN)] * M
        + [_full_spec((n_inp, ci, HIDDEN)),
           _full_spec((M, HIDDEN, HIDDEN)),
           _full_spec((1, HIDDEN)),
           _row_spec(HIDDEN),
           _row_spec(HIDDEN)]
    )
    return pl.pallas_call(
        functools.partial(_cand_body, n_inp),
        grid=(R // BR,),
        in_specs=in_specs,
        out_specs=_row_spec(HIDDEN),
        out_shape=jax.ShapeDtypeStruct((R, HIDDEN), jnp.float32),
    )


def _to_slices(x_bni, in_part, fc):
    bpg = fc // in_part
    s = B // bpg
    x = jnp.pad(x_bni, ((0, 0), (0, N_PAD - N), (0, 0)))
    x = x.reshape(s, bpg, N_PAD, in_part)
    return jnp.transpose(x, (0, 2, 1, 3)).reshape(s, N_PAD, fc)


def _realign_l0(t_sl):
    x = t_sl[0, :, :B * INPUT_DIM].reshape(N_PAD, 4, 4, INPUT_DIM)
    x = jnp.transpose(x, (1, 0, 2, 3))
    return x.reshape(R, INPUT_DIM)


def _from_rows(h_rows):
    x = h_rows.reshape(4, N_PAD, 4, HIDDEN)[:, :N]
    x = jnp.transpose(x, (0, 2, 1, 3))
    return x.reshape(B, N * HIDDEN)


def _eff_weights(w, in0):
    w3 = w.reshape(in0, M, w.shape[1])
    return jnp.stack(
        [w3[:, 0] - w3[:, 2], w3[:, 1] - w3[:, 4], w3[:, 2], w3[:, 3],
         w3[:, 4]], axis=1)


def kernel(inputs, hidden_state, edge_src, edge_dst, edge_w,
           Wg0, bg0, Wc0, bc0, Wg1, bg1, Wc1, bc1):
    deg_out = jax.ops.segment_sum(edge_w, edge_src, num_segments=N)
    inv_out = jnp.where(deg_out > 0, 1.0 / deg_out, 0.0)
    val1 = edge_w * inv_out[edge_src]
    deg_in = jax.ops.segment_sum(edge_w, edge_dst, num_segments=N)
    inv_in = jnp.where(deg_in > 0, 1.0 / deg_in, 0.0)
    val2 = edge_w * inv_in[edge_dst]

    pad = E_PAD - E

    def tile_edges(a):
        return jnp.pad(a, (0, pad)).reshape(NUM_TILES, NB, KB)

    def tile_vals(a):
        return jnp.pad(a, (0, pad)).reshape(NUM_TILES, NB * KB)

    src_t = tile_edges(edge_src)
    dst_t = tile_edges(edge_dst)
    v1 = tile_vals(val1)
    v2 = tile_vals(val2)
    zeros128 = jnp.zeros((N_PAD, 128), jnp.float32)

    def chain(x_sl, zeros):
        t1, t2p, t3, t4p = _make_chain(x_sl.shape[0])(
            x_sl, src_t, dst_t, v1, v2, zeros)
        return [x_sl, t1, t2p, t3, t4p]

    x_in = jnp.pad(inputs.reshape(B, N, INPUT_DIM),
                   ((0, 0), (0, N_PAD - N), (0, 0)))
    x_in = jnp.transpose(x_in, (1, 0, 2)).reshape(1, N_PAD, B * INPUT_DIM)
    x_in = jnp.pad(x_in, ((0, 1), (0, 0), (0, 128 - B * INPUT_DIM)))
    inp_terms0 = chain(x_in, zeros128)
    ti0 = jnp.concatenate([_realign_l0(t) for t in inp_terms0], axis=1)
    ti0 = jnp.pad(ti0, ((0, 0), (0, 16 - M * INPUT_DIM)))

    params = [(Wg0, bg0, Wc0, bc0, INPUT_DIM + HIDDEN),
              (Wg1, bg1, Wc1, bc1, 2 * HIDDEN)]
    output_rows = None
    new_hidden = []
    for layer in range(NUM_LAYERS):
        wg, bg, wc, bc, in0 = params[layer]
        ini = in0 - HIDDEN
        hx_sl = _to_slices(
            hidden_state[layer].reshape(B, N, HIDDEN), HIDDEN, 128)
        hx_rows = hx_sl.reshape(R, HIDDEN)

        if layer == 0:
            wge = _eff_weights(wg, in0)
            wce = _eff_weights(wc, in0)
            wi_g = jnp.transpose(wge[:ini], (1, 0, 2)).reshape(M * ini, -1)
            wi_g = jnp.pad(wi_g, ((0, 16 - M * ini), (0, 0)))[None]
            wi_c = jnp.transpose(wce[:ini], (1, 0, 2)).reshape(M * ini, -1)
            wi_c = jnp.pad(wi_c, ((0, 16 - M * ini), (0, 0)))[None]
            ti_list = [ti0]
            n_inp, ci = 1, 16
        else:
            wge = _eff_weights(wg, in0)
            wce = _eff_weights(wc, in0)
            wi_g = jnp.transpose(wge[:ini], (1, 0, 2))
            wi_c = jnp.transpose(wce[:ini], (1, 0, 2))
            inp_terms = chain(output_rows.reshape(4, N_PAD, 128), zeros128)
            ti_list = [t.reshape(R, HIDDEN) for t in inp_terms]
            n_inp, ci = M, HIDDEN

        wh_g = jnp.transpose(wge[ini:], (1, 0, 2))
        wh_c = jnp.transpose(wce[ini:], (1, 0, 2))

        h_terms = chain(hx_sl, zeros128)
        th_list = [t.reshape(R, HIDDEN) for t in h_terms]

        rh_rows, u_rows = _make_gate(n_inp, ci)(
            *ti_list, *th_list, wi_g, wh_g, bg.reshape(1, -1), hx_rows)

        rh_terms = chain(rh_rows.reshape(4, N_PAD, 128), zeros128)
        trh_list = [t.reshape(R, HIDDEN) for t in rh_terms]

        hn_rows = _make_cand(n_inp, ci)(
            *ti_list, *trh_list, wi_c, wh_c, bc.reshape(1, -1),
            u_rows, hx_rows)

        new_hidden.append(_from_rows(hn_rows))
        output_rows = hn_rows

    output = new_hidden[-1]
    return (output, jnp.stack(new_hidden))


# speedup vs baseline: 2.3890x; 1.0508x over previous
import functools

import jax
import jax.numpy as jnp
from jax import lax
from jax.experimental import pallas as pl
from jax.experimental.pallas import tpu as pltpu
from jax.experimental.pallas import tpu_sc as plsc

N = 10000
E = 160000
B = 16
INPUT_DIM = 2
HIDDEN = 32
NUM_LAYERS = 2
K = 2
M = 2 * K + 1

NUM_TILES = 16
NUM_CORES = 2
KB = 128
NB = 2 * -(-(E // NUM_TILES) // (2 * KB))
EP_TILE = NB * KB
E_PAD = EP_TILE * NUM_TILES

N_PAD = 10240
STRIPE = N_PAD // NUM_TILES


def _splat(i):
    return jnp.full((16,), i, jnp.int32)

FC = 128

R = N_PAD * B
BR = 2048


def _chain_body(S, x_hbm, combo_hbm, v1_hbm, v2_hbm, zeros_hbm,
                t1_hbm, t2_hbm, t3_hbm, t4_hbm,
                combo_v, valb0_v, valb1_v, buf0_v, buf1_v,
                gat0_v, gat1_v, sct0_v, sct1_v, acc_sh, sem0, sem1):
    c = lax.axis_index("c")
    s = lax.axis_index("s")
    spc = S // NUM_CORES
    pltpu.sync_copy(combo_hbm.at[s], combo_v)

    def unpack_idx(b, swap, gat32, sct32):
        for g in range(KB // 16):
            sl16 = pl.ds(g * 16, 16)
            w = combo_v[b, sl16]
            lo = w & 0xFFFF
            hi = (w >> 16) & 0xFFFF
            gat32[sl16] = hi if swap else lo
            sct32[sl16] = lo if swap else hi

    v1_t = v1_hbm.at[s]
    v2_t = v2_hbm.at[s]
    stages = [
        (x_hbm, False, v1_t, False, t1_hbm),
        (t1_hbm, False, v1_t, True, t2_hbm),
        (t1_hbm, True, v2_t, False, t3_hbm),
        (t3_hbm, True, v2_t, True, t4_hbm),
    ]
    for p in range(spc):
        sl = c * spc + p
        for x_in, swap, v_t, dbl, out_hbm in stages:
            pltpu.sync_copy(zeros_hbm.at[pl.ds(s * STRIPE, STRIPE)],
                            acc_sh.at[pl.ds(s * STRIPE, STRIPE)])
            plsc.subcore_barrier()
            x_sl = x_in.at[sl]

            def start(b, buf, valb, gat32, sct32, sm):
                unpack_idx(b, swap, gat32, sct32)
                pltpu.async_copy(x_sl.at[gat32], buf, sm)
                pltpu.async_copy(v_t.at[pl.ds(b * KB, KB)], valb, sm)

            def finish(b, buf, valb, gat32, sct32, sm):
                pltpu.make_async_copy(x_sl.at[gat32], buf, sm).wait()
                pltpu.make_async_copy(
                    v_t.at[pl.ds(b * KB, KB)], valb, sm).wait()

                def group_body(g, carry2):
                    vk = valb[pl.ds(g * 16, 16)]
                    if dbl:
                        vk = vk + vk
                    for rr in range(16):
                        v = vk.at[_splat(rr)].get(mode="promise_in_bounds")
                        r = g * 16 + rr
                        for j in range(FC // 16):
                            sl16 = pl.ds(j * 16, 16)
                            buf[r, sl16] = buf[r, sl16] * v
                    return carry2

                lax.fori_loop(0, KB // 16, group_body, 0)
                pltpu.sync_copy(buf, acc_sh.at[sct32], add=True)

            start(0, buf0_v, valb0_v, gat0_v, sct0_v, sem0)

            def pair_body(p, carry):
                b0 = 2 * p
                b1 = b0 + 1
                start(b1, buf1_v, valb1_v, gat1_v, sct1_v, sem1)
                finish(b0, buf0_v, valb0_v, gat0_v, sct0_v, sem0)

                @pl.when(b0 + 2 < NB)
                def _prefetch():
                    start(b0 + 2, buf0_v, valb0_v, gat0_v, sct0_v, sem0)

                finish(b1, buf1_v, valb1_v, gat1_v, sct1_v, sem1)
                return carry

            lax.fori_loop(0, NB // 2, pair_body, 0)
            plsc.subcore_barrier()
            pltpu.sync_copy(acc_sh.at[pl.ds(s * STRIPE, STRIPE)],
                            out_hbm.at[sl, pl.ds(s * STRIPE, STRIPE)])
            plsc.subcore_barrier()


@functools.cache
def _make_chain(S):
    body = functools.partial(_chain_body, S)
    term = jax.ShapeDtypeStruct((S, N_PAD, FC), jnp.float32)
    return pl.kernel(
        body,
        out_type=(term, term, term, term),
        mesh=plsc.VectorSubcoreMesh(core_axis_name="c", subcore_axis_name="s",
                                    num_cores=NUM_CORES,
                                    num_subcores=NUM_TILES),
        scratch_types=[
            pltpu.VMEM((NB, KB), jnp.int32),
            pltpu.VMEM((KB,), jnp.float32),
            pltpu.VMEM((KB,), jnp.float32),
            pltpu.VMEM((KB, FC), jnp.float32),
            pltpu.VMEM((KB, FC), jnp.float32),
            pltpu.VMEM((KB,), jnp.int32),
            pltpu.VMEM((KB,), jnp.int32),
            pltpu.VMEM((KB,), jnp.int32),
            pltpu.VMEM((KB,), jnp.int32),
            pltpu.VMEM_SHARED((N_PAD, FC), jnp.float32),
            pltpu.SemaphoreType.DMA,
            pltpu.SemaphoreType.DMA,
        ],
    )


def _gate_body(n_inp, *refs):
    ti = refs[0:n_inp]
    th = refs[n_inp:n_inp + M]
    wi, wh, bg, h = refs[n_inp + M:n_inp + M + 4]
    rh_o, u_o = refs[n_inp + M + 4:]
    acc = jnp.zeros((BR, 2 * HIDDEN), jnp.float32) + bg[...]
    for m in range(n_inp):
        acc += jnp.dot(ti[m][...], wi[m], preferred_element_type=jnp.float32)
    for m in range(M):
        acc += jnp.dot(th[m][...], wh[m], preferred_element_type=jnp.float32)
    value = jax.nn.sigmoid(acc)
    r = value[:, :HIDDEN]
    u = value[:, HIDDEN:]
    rh_o[...] = r * h[...]
    u_o[...] = u


def _cand_body(n_inp, *refs):
    ti = refs[0:n_inp]
    trh = refs[n_inp:n_inp + M]
    wi, wh, bc, u, hx = refs[n_inp + M:n_inp + M + 5]
    hn_o = refs[n_inp + M + 5]
    acc = jnp.zeros((BR, HIDDEN), jnp.float32) + bc[...]
    for m in range(n_inp):
        acc += jnp.dot(ti[m][...], wi[m], preferred_element_type=jnp.float32)
    for m in range(M):
        acc += jnp.dot(trh[m][...], wh[m], preferred_element_type=jnp.float32)
    c = jnp.tanh(acc)
    uu = u[...]
    hn_o[...] = uu * hx[...] + (1.0 - uu) * c


def _row_spec(cols):
    return pl.BlockSpec((BR, cols), lambda i: (i, 0))


def _full_spec(shape):
    nd = len(shape)
    return pl.BlockSpec(shape, lambda i: (0,) * nd)


@functools.cache
def _make_gate(n_inp, ci):
    in_specs = (
        [_row_spec(ci)] * n_inp
        + [_row_spec(HIDDEN)] * M
        + [_full_spec((n_inp, ci, 2 * HIDDEN)),
           _full_spec((M, HIDDEN, 2 * HIDDEN)),
           _full_spec((1, 2 * HIDDEN)),
           _row_spec(HIDDEN)]
    )
    return pl.pallas_call(
        functools.partial(_gate_body, n_inp),
        grid=(R // BR,),
        in_specs=in_specs,
        out_specs=[_row_spec(HIDDEN)] * 2,
        out_shape=[jax.ShapeDtypeStruct((R, HIDDEN), jnp.float32)] * 2,
    )


@functools.cache
def _make_cand(n_inp, ci):
    in_specs = (
        [_row_spec(ci)] * n_inp
        + [_row_spec(HIDDEN)] * M
        + [_full_spec((n_inp, ci, HIDDEN)),
           _full_spec((M, HIDDEN, HIDDEN)),
           _full_spec((1, HIDDEN)),
           _row_spec(HIDDEN),
           _row_spec(HIDDEN)]
    )
    return pl.pallas_call(
        functools.partial(_cand_body, n_inp),
        grid=(R // BR,),
        in_specs=in_specs,
        out_specs=_row_spec(HIDDEN),
        out_shape=jax.ShapeDtypeStruct((R, HIDDEN), jnp.float32),
    )


def _to_slices(x_bni, in_part, fc):
    bpg = fc // in_part
    s = B // bpg
    x = jnp.pad(x_bni, ((0, 0), (0, N_PAD - N), (0, 0)))
    x = x.reshape(s, bpg, N_PAD, in_part)
    return jnp.transpose(x, (0, 2, 1, 3)).reshape(s, N_PAD, fc)


def _realign_l0(t_sl):
    x = t_sl[0, :, :B * INPUT_DIM].reshape(N_PAD, 4, 4, INPUT_DIM)
    x = jnp.transpose(x, (1, 0, 2, 3))
    return x.reshape(R, INPUT_DIM)


def _from_rows(h_rows):
    x = h_rows.reshape(4, N_PAD, 4, HIDDEN)[:, :N]
    x = jnp.transpose(x, (0, 2, 1, 3))
    return x.reshape(B, N * HIDDEN)


def _eff_weights(w, in0):
    w3 = w.reshape(in0, M, w.shape[1])
    return jnp.stack(
        [w3[:, 0] - w3[:, 2], w3[:, 1] - w3[:, 4], w3[:, 2], w3[:, 3],
         w3[:, 4]], axis=1)


def kernel(inputs, hidden_state, edge_src, edge_dst, edge_w,
           Wg0, bg0, Wc0, bc0, Wg1, bg1, Wc1, bc1):
    deg_out = jax.ops.segment_sum(edge_w, edge_src, num_segments=N)
    inv_out = jnp.where(deg_out > 0, 1.0 / deg_out, 0.0)
    val1 = edge_w * inv_out[edge_src]
    deg_in = jax.ops.segment_sum(edge_w, edge_dst, num_segments=N)
    inv_in = jnp.where(deg_in > 0, 1.0 / deg_in, 0.0)
    val2 = edge_w * inv_in[edge_dst]

    pad = E_PAD - E

    def tile_edges(a):
        return jnp.pad(a, (0, pad)).reshape(NUM_TILES, NB, KB)

    def tile_vals(a):
        return jnp.pad(a, (0, pad)).reshape(NUM_TILES, NB * KB)

    combo_t = tile_edges(edge_src | (edge_dst << 16))
    v1 = tile_vals(val1)
    v2 = tile_vals(val2)
    zeros128 = jnp.zeros((N_PAD, 128), jnp.float32)

    def chain(x_sl, zeros):
        t1, t2p, t3, t4p = _make_chain(x_sl.shape[0])(
            x_sl, combo_t, v1, v2, zeros)
        return [x_sl, t1, t2p, t3, t4p]

    x_in = jnp.pad(inputs.reshape(B, N, INPUT_DIM),
                   ((0, 0), (0, N_PAD - N), (0, 0)))
    x_in = jnp.transpose(x_in, (1, 0, 2)).reshape(1, N_PAD, B * INPUT_DIM)
    x_in = jnp.pad(x_in, ((0, 1), (0, 0), (0, 128 - B * INPUT_DIM)))
    inp_terms0 = chain(x_in, zeros128)
    ti0 = jnp.concatenate([_realign_l0(t) for t in inp_terms0], axis=1)
    ti0 = jnp.pad(ti0, ((0, 0), (0, 16 - M * INPUT_DIM)))

    params = [(Wg0, bg0, Wc0, bc0, INPUT_DIM + HIDDEN),
              (Wg1, bg1, Wc1, bc1, 2 * HIDDEN)]
    output_rows = None
    new_hidden = []
    for layer in range(NUM_LAYERS):
        wg, bg, wc, bc, in0 = params[layer]
        ini = in0 - HIDDEN
        hx_sl = _to_slices(
            hidden_state[layer].reshape(B, N, HIDDEN), HIDDEN, 128)
        hx_rows = hx_sl.reshape(R, HIDDEN)

        if layer == 0:
            wge = _eff_weights(wg, in0)
            wce = _eff_weights(wc, in0)
            wi_g = jnp.transpose(wge[:ini], (1, 0, 2)).reshape(M * ini, -1)
            wi_g = jnp.pad(wi_g, ((0, 16 - M * ini), (0, 0)))[None]
            wi_c = jnp.transpose(wce[:ini], (1, 0, 2)).reshape(M * ini, -1)
            wi_c = jnp.pad(wi_c, ((0, 16 - M * ini), (0, 0)))[None]
            ti_list = [ti0]
            n_inp, ci = 1, 16
        else:
            wge = _eff_weights(wg, in0)
            wce = _eff_weights(wc, in0)
            wi_g = jnp.transpose(wge[:ini], (1, 0, 2))
            wi_c = jnp.transpose(wce[:ini], (1, 0, 2))
            inp_terms = chain(output_rows.reshape(4, N_PAD, 128), zeros128)
            ti_list = [t.reshape(R, HIDDEN) for t in inp_terms]
            n_inp, ci = M, HIDDEN

        wh_g = jnp.transpose(wge[ini:], (1, 0, 2))
        wh_c = jnp.transpose(wce[ini:], (1, 0, 2))

        h_terms = chain(hx_sl, zeros128)
        th_list = [t.reshape(R, HIDDEN) for t in h_terms]

        rh_rows, u_rows = _make_gate(n_inp, ci)(
            *ti_list, *th_list, wi_g, wh_g, bg.reshape(1, -1), hx_rows)

        rh_terms = chain(rh_rows.reshape(4, N_PAD, 128), zeros128)
        trh_list = [t.reshape(R, HIDDEN) for t in rh_terms]

        hn_rows = _make_cand(n_inp, ci)(
            *ti_list, *trh_list, wi_c, wh_c, bc.reshape(1, -1),
            u_rows, hx_rows)

        new_hidden.append(_from_rows(hn_rows))
        output_rows = hn_rows

    output = new_hidden[-1]
    return (output, jnp.stack(new_hidden))
